# baseline (speedup 1.0000x reference)
import functools

import numpy as np
import jax
import jax.numpy as jnp
from jax import lax
from jax.experimental import pallas as pl
from jax.experimental.pallas import tpu as pltpu
from jax.experimental.pallas import tpu_sc as plsc

_B = 16384
_ND = 13
_NT = 26
_V = 100000
_D = 32
_N1 = _NT + 1

_NC = 2
_NS = 16
_NW = _NC * _NS
_ROWS = _B * _NT
_RPW = _ROWS // _NW
_CHUNK = 1664
_NCHUNK = _RPW // _CHUNK
def _sc_gather(table, flat_idx):
    mesh = plsc.VectorSubcoreMesh(core_axis_name="c", subcore_axis_name="s")

    @functools.partial(
        pl.kernel,
        out_type=jax.ShapeDtypeStruct((_ROWS, _D), jnp.float32),
        mesh=mesh,
        scratch_types=[
            pltpu.VMEM((_CHUNK,), jnp.int32),
            pltpu.VMEM((_CHUNK, _D), jnp.float32),
            pltpu.SemaphoreType.DMA,
        ],
        compiler_params=pltpu.CompilerParams(use_tc_tiling_on_sc=False),
    )
    def k(table_hbm, idx_hbm, out_hbm, idx_v, rows_v, sem):
        wid = lax.axis_index("s") * _NC + lax.axis_index("c")
        base = wid * _RPW
        for i in range(_NCHUNK):
            off = base + i * _CHUNK
            pltpu.sync_copy(idx_hbm.at[pl.ds(off, _CHUNK)], idx_v)
            pltpu.async_copy(table_hbm.at[idx_v], rows_v, sem).wait()
            pltpu.sync_copy(rows_v, out_hbm.at[pl.ds(off, _CHUNK)])

    return k(table, flat_idx)


_Q = _V // 4


def _table_shuffle(embT2):
    def _in_copy(x_hbm, xv, sems, t, slot):
        return pltpu.make_async_copy(
            x_hbm.at[pl.ds(t * _D, _D), :], xv.at[slot], sems.at[slot])

    def _out_copy(o_hbm, ov, sems, t):
        return pltpu.make_async_copy(
            ov, o_hbm.at[pl.ds(t * _Q, _Q), :], sems.at[2])

    def body(x_hbm, o_hbm, xv, ov, sems):
        t = pl.program_id(0)
        slot = lax.rem(t, 2)

        @pl.when(t == 0)
        def _prologue():
            _in_copy(x_hbm, xv, sems, 0, 0).start()

        @pl.when(t < _NT - 1)
        def _prefetch():
            _in_copy(x_hbm, xv, sems, t + 1, 1 - slot).start()

        _in_copy(x_hbm, xv, sems, t, slot).wait()

        @pl.when(t >= 1)
        def _drain_prev():
            _out_copy(o_hbm, ov, sems, t - 1).wait()

        eye = jnp.eye(_D, dtype=jnp.float32)
        for j in range(4):
            xth = jax.lax.dot_general(
                xv[slot, :, j * _Q:(j + 1) * _Q], eye,
                (((0,), (0,)), ((), ())),
                preferred_element_type=jnp.float32)
            ov[:, j * _D:(j + 1) * _D] = xth
        _out_copy(o_hbm, ov, sems, t).start()

        @pl.when(t >= _NT - 1)
        def _epilogue():
            _out_copy(o_hbm, ov, sems, t).wait()

    return pl.pallas_call(
        body,
        grid=(_NT,),
        in_specs=[pl.BlockSpec(memory_space=pl.ANY)],
        out_specs=pl.BlockSpec(memory_space=pl.ANY),
        out_shape=jax.ShapeDtypeStruct((_NT * _Q, 4 * _D), jnp.float32),
        scratch_shapes=[
            pltpu.VMEM((2, _D, _V), jnp.float32),
            pltpu.VMEM((_Q, 4 * _D), jnp.float32),
            pltpu.SemaphoreType.DMA((3,)),
        ],
        compiler_params=pltpu.CompilerParams(
            dimension_semantics=("arbitrary",),
            vmem_limit_bytes=60 * 1024 * 1024,
            fuse_transposed_lhs_in_matmul=True,
        ),
    )(embT2)


_BB = 256


def _tc_body(numT, ly, bw0, bb0, bw1, bb1, bw2, bb2,
             tw0p, tb0, tw1, tb1, tw2, tb2, tw3, tb3, tw4, tb4, out):
    f32 = jnp.float32

    def mm(w, x):
        return jnp.dot(w, x, preferred_element_type=f32)

    x = numT[...]
    x = jnp.maximum(mm(bw0[...], x) + bb0[...], 0.0)
    x = jnp.maximum(mm(bw1[...], x) + bb1[...], 0.0)
    bot = jnp.maximum(mm(bw2[...], x) + bb2[...], 0.0)

    lyT = jnp.transpose(ly[...])
    ly3 = lyT.reshape(_NT, _D, _BB)

    pieces = [bot, jnp.sum(ly3 * bot[None, :, :], axis=1)]
    for o in range(1, _NT):
        kk = _NT - o
        v = ly3[o:, :, :] * ly3[:kk, :, :]
        pieces.append(jnp.sum(v, axis=1))
    t = jnp.concatenate(pieces, axis=0)

    t = jnp.maximum(mm(tw0p[...], t) + tb0[...], 0.0)
    t = jnp.maximum(mm(tw1[...], t) + tb1[...], 0.0)
    t = jnp.maximum(mm(tw2[...], t) + tb2[...], 0.0)
    t = jnp.maximum(mm(tw3[...], t) + tb3[...], 0.0)
    out[...] = jax.nn.sigmoid(mm(tw4[...], t) + tb4[...])


def _tc_forward(numT, ly2d, bw0, bb0, bw1, bb1, bw2, bb2,
                tw0p, tb0, tw1, tb1, tw2, tb2, tw3, tb3, tw4, tb4):
    nblk = _B // _BB

    def full(a):
        return pl.BlockSpec(a.shape, lambda i: (0,) * a.ndim)

    in_specs = [
        pl.BlockSpec((_ND, _BB), lambda i: (0, i)),
        pl.BlockSpec((_BB, _NT * _D), lambda i: (i, 0)),
    ] + [full(a) for a in (bw0, bb0, bw1, bb1, bw2, bb2,
                           tw0p, tb0, tw1, tb1, tw2, tb2, tw3, tb3, tw4, tb4)]

    return pl.pallas_call(
        _tc_body,
        grid=(nblk,),
        in_specs=in_specs,
        out_specs=pl.BlockSpec((1, _BB), lambda i: (0, i)),
        out_shape=jax.ShapeDtypeStruct((1, _B), jnp.float32),
        compiler_params=pltpu.CompilerParams(
            dimension_semantics=("arbitrary",),
        ),
    )(numT, ly2d, bw0, bb0, bw1, bb1, bw2, bb2,
      tw0p, tb0, tw1, tb1, tw2, tb2, tw3, tb3, tw4, tb4)


def _offset_perm():
    r_idx, c_idx = np.tril_indices(_N1, -1)
    pos = {(int(r), int(c)): j for j, (r, c) in enumerate(zip(r_idx, c_idx))}
    order = [pos[(o, 0)] for o in range(1, _N1)]
    order += [pos[(j + o + 1, j + 1)]
              for o in range(1, _NT) for j in range(_NT - o)]
    return np.array(order, dtype=np.int32)


def kernel(numerical_input, categorical_inputs, emb,
           bw0, bb0, bw1, bb1, bw2, bb2,
           tw0, tb0, tw1, tb1, tw2, tb2, tw3, tb3, tw4, tb4):
    embT2 = emb.transpose(0, 2, 1).reshape(_NT * _D, _V)
    cat = categorical_inputs
    shuf = 4 * (cat % _Q) + cat // _Q
    flat_idx = (shuf
                + (jnp.arange(_NT, dtype=jnp.int32) * _V)[None, :]).reshape(-1)
    numT = numerical_input.T
    perm = _offset_perm()
    tw0p = jnp.concatenate([tw0[:, :_D], tw0[:, _D + perm]], axis=1)

    def col(b):
        return b.reshape(-1, 1)

    table = _table_shuffle(embT2).reshape(_NT * _V, _D)

    ly = _sc_gather(table, flat_idx)
    ly2d = ly.reshape(_B, _NT * _D)

    outT = _tc_forward(numT, ly2d, bw0, col(bb0), bw1, col(bb1), bw2, col(bb2),
                       tw0p, col(tb0), tw1, col(tb1), tw2, col(tb2),
                       tw3, col(tb3), tw4, col(tb4))
    return outT.T


# speedup vs baseline: 1.0563x; 1.0563x over previous
import functools

import numpy as np
import jax
import jax.numpy as jnp
from jax import lax
from jax.experimental import pallas as pl
from jax.experimental.pallas import tpu as pltpu
from jax.experimental.pallas import tpu_sc as plsc

_B = 16384
_ND = 13
_NT = 26
_V = 100000
_D = 32
_N1 = _NT + 1

_NC = 2
_NS = 16
_NW = _NC * _NS
_ROWS = _B * _NT
_RPW = _ROWS // _NW
_CHUNK = 1664
_NCHUNK = _RPW // _CHUNK
def _sc_gather(table, flat_idx):
    mesh = plsc.VectorSubcoreMesh(core_axis_name="c", subcore_axis_name="s")

    @functools.partial(
        pl.kernel,
        out_type=jax.ShapeDtypeStruct((_ROWS, _D), jnp.float32),
        mesh=mesh,
        scratch_types=[
            pltpu.VMEM((_CHUNK,), jnp.int32),
            pltpu.VMEM((_CHUNK, _D), jnp.float32),
            pltpu.SemaphoreType.DMA,
        ],
        compiler_params=pltpu.CompilerParams(use_tc_tiling_on_sc=False),
    )
    def k(table_hbm, idx_hbm, out_hbm, idx_v, rows_v, sem):
        wid = lax.axis_index("s") * _NC + lax.axis_index("c")
        base = wid * _RPW
        for i in range(_NCHUNK):
            off = base + i * _CHUNK
            pltpu.sync_copy(idx_hbm.at[pl.ds(off, _CHUNK)], idx_v)
            pltpu.async_copy(table_hbm.at[idx_v], rows_v, sem).wait()
            pltpu.sync_copy(rows_v, out_hbm.at[pl.ds(off, _CHUNK)])

    return k(table, flat_idx)


_Q = _V // 4


def _table_shuffle(embT2):
    def _in_copy(x_hbm, xv, sems, t, slot):
        return pltpu.make_async_copy(
            x_hbm.at[pl.ds(t * _D, _D), :], xv.at[slot], sems.at[slot])

    def _out_copy(o_hbm, ov, sems, t):
        return pltpu.make_async_copy(
            ov, o_hbm.at[pl.ds(t * _Q, _Q), :], sems.at[2])

    def body(x_hbm, o_hbm, xv, ov, sems):
        t = pl.program_id(0)
        slot = lax.rem(t, 2)

        @pl.when(t == 0)
        def _prologue():
            _in_copy(x_hbm, xv, sems, 0, 0).start()

        @pl.when(t < _NT - 1)
        def _prefetch():
            _in_copy(x_hbm, xv, sems, t + 1, 1 - slot).start()

        _in_copy(x_hbm, xv, sems, t, slot).wait()

        @pl.when(t >= 1)
        def _drain_prev():
            _out_copy(o_hbm, ov, sems, t - 1).wait()

        eye = jnp.eye(_D, dtype=jnp.float32)
        for j in range(4):
            xth = jax.lax.dot_general(
                xv[slot, :, j * _Q:(j + 1) * _Q], eye,
                (((0,), (0,)), ((), ())),
                preferred_element_type=jnp.float32)
            ov[:, j * _D:(j + 1) * _D] = xth
        _out_copy(o_hbm, ov, sems, t).start()

        @pl.when(t >= _NT - 1)
        def _epilogue():
            _out_copy(o_hbm, ov, sems, t).wait()

    return pl.pallas_call(
        body,
        grid=(_NT,),
        in_specs=[pl.BlockSpec(memory_space=pl.ANY)],
        out_specs=pl.BlockSpec(memory_space=pl.ANY),
        out_shape=jax.ShapeDtypeStruct((_NT * _Q, 4 * _D), jnp.float32),
        scratch_shapes=[
            pltpu.VMEM((2, _D, _V), jnp.float32),
            pltpu.VMEM((_Q, 4 * _D), jnp.float32),
            pltpu.SemaphoreType.DMA((3,)),
        ],
        compiler_params=pltpu.CompilerParams(
            dimension_semantics=("arbitrary",),
            vmem_limit_bytes=60 * 1024 * 1024,
            fuse_transposed_lhs_in_matmul=True,
        ),
    )(embT2)


_BB = 512


def _tc_body(numT, ly, bw0, bb0, bw1, bb1, bw2, bb2,
             tw0p, tb0, tw1, tb1, tw2, tb2, tw3, tb3, tw4, tb4, out):
    f32 = jnp.float32

    def mm(w, x):
        return jnp.dot(w, x, preferred_element_type=f32)

    x = numT[...]
    x = jnp.maximum(mm(bw0[...], x) + bb0[...], 0.0)
    x = jnp.maximum(mm(bw1[...], x) + bb1[...], 0.0)
    bot = jnp.maximum(mm(bw2[...], x) + bb2[...], 0.0)

    lyT = jnp.transpose(ly[...])
    ly3 = lyT.reshape(_NT, _D, _BB)

    pieces = [bot, jnp.sum(ly3 * bot[None, :, :], axis=1)]
    for o in range(1, _NT):
        kk = _NT - o
        v = ly3[o:, :, :] * ly3[:kk, :, :]
        pieces.append(jnp.sum(v, axis=1))
    t = jnp.concatenate(pieces, axis=0)

    t = jnp.maximum(mm(tw0p[...], t) + tb0[...], 0.0)
    t = jnp.maximum(mm(tw1[...], t) + tb1[...], 0.0)
    t = jnp.maximum(mm(tw2[...], t) + tb2[...], 0.0)
    t = jnp.maximum(mm(tw3[...], t) + tb3[...], 0.0)
    out[...] = jax.nn.sigmoid(mm(tw4[...], t) + tb4[...])


def _tc_forward(numT, ly2d, bw0, bb0, bw1, bb1, bw2, bb2,
                tw0p, tb0, tw1, tb1, tw2, tb2, tw3, tb3, tw4, tb4):
    nblk = _B // _BB

    def full(a):
        return pl.BlockSpec(a.shape, lambda i: (0,) * a.ndim)

    in_specs = [
        pl.BlockSpec((_ND, _BB), lambda i: (0, i)),
        pl.BlockSpec((_BB, _NT * _D), lambda i: (i, 0)),
    ] + [full(a) for a in (bw0, bb0, bw1, bb1, bw2, bb2,
                           tw0p, tb0, tw1, tb1, tw2, tb2, tw3, tb3, tw4, tb4)]

    return pl.pallas_call(
        _tc_body,
        grid=(nblk,),
        in_specs=in_specs,
        out_specs=pl.BlockSpec((1, _BB), lambda i: (0, i)),
        out_shape=jax.ShapeDtypeStruct((1, _B), jnp.float32),
        compiler_params=pltpu.CompilerParams(
            dimension_semantics=("arbitrary",),
        ),
    )(numT, ly2d, bw0, bb0, bw1, bb1, bw2, bb2,
      tw0p, tb0, tw1, tb1, tw2, tb2, tw3, tb3, tw4, tb4)


def _offset_perm():
    r_idx, c_idx = np.tril_indices(_N1, -1)
    pos = {(int(r), int(c)): j for j, (r, c) in enumerate(zip(r_idx, c_idx))}
    order = [pos[(o, 0)] for o in range(1, _N1)]
    order += [pos[(j + o + 1, j + 1)]
              for o in range(1, _NT) for j in range(_NT - o)]
    return np.array(order, dtype=np.int32)


def kernel(numerical_input, categorical_inputs, emb,
           bw0, bb0, bw1, bb1, bw2, bb2,
           tw0, tb0, tw1, tb1, tw2, tb2, tw3, tb3, tw4, tb4):
    embT2 = emb.transpose(0, 2, 1).reshape(_NT * _D, _V)
    cat = categorical_inputs
    shuf = 4 * (cat % _Q) + cat // _Q
    flat_idx = (shuf
                + (jnp.arange(_NT, dtype=jnp.int32) * _V)[None, :]).reshape(-1)
    numT = numerical_input.T
    perm = _offset_perm()
    tw0p = jnp.concatenate([tw0[:, :_D], tw0[:, _D + perm]], axis=1)

    def col(b):
        return b.reshape(-1, 1)

    table = _table_shuffle(embT2).reshape(_NT * _V, _D)

    ly = _sc_gather(table, flat_idx)
    ly2d = ly.reshape(_B, _NT * _D)

    outT = _tc_forward(numT, ly2d, bw0, col(bb0), bw1, col(bb1), bw2, col(bb2),
                       tw0p, col(tb0), tw1, col(tb1), tw2, col(tb2),
                       tw3, col(tb3), tw4, col(tb4))
    return outT.T


# speedup vs baseline: 1.0940x; 1.0356x over previous
import functools

import numpy as np
import jax
import jax.numpy as jnp
from jax import lax
from jax.experimental import pallas as pl
from jax.experimental.pallas import tpu as pltpu
from jax.experimental.pallas import tpu_sc as plsc

_B = 16384
_ND = 13
_NT = 26
_V = 100000
_D = 32
_N1 = _NT + 1

_NC = 2
_NS = 16
_NW = _NC * _NS
_ROWS = _B * _NT
_RPW = _ROWS // _NW
_CHUNK = 1664
_NCHUNK = _RPW // _CHUNK
def _sc_gather(table, flat_idx):
    mesh = plsc.VectorSubcoreMesh(core_axis_name="c", subcore_axis_name="s")

    @functools.partial(
        pl.kernel,
        out_type=jax.ShapeDtypeStruct((_ROWS, _D), jnp.float32),
        mesh=mesh,
        scratch_types=[
            pltpu.VMEM((_CHUNK,), jnp.int32),
            pltpu.VMEM((_CHUNK, _D), jnp.float32),
            pltpu.SemaphoreType.DMA,
        ],
        compiler_params=pltpu.CompilerParams(use_tc_tiling_on_sc=False),
    )
    def k(table_hbm, idx_hbm, out_hbm, idx_v, rows_v, sem):
        wid = lax.axis_index("s") * _NC + lax.axis_index("c")
        base = wid * _RPW
        for i in range(_NCHUNK):
            off = base + i * _CHUNK
            pltpu.sync_copy(idx_hbm.at[pl.ds(off, _CHUNK)], idx_v)
            pltpu.async_copy(table_hbm.at[idx_v], rows_v, sem).wait()
            pltpu.sync_copy(rows_v, out_hbm.at[pl.ds(off, _CHUNK)])

    return k(table, flat_idx)


_Q = _V // 4


def _table_shuffle(embT2):
    def _in_copy(x_hbm, xv, sems, t, slot):
        return pltpu.make_async_copy(
            x_hbm.at[pl.ds(t * _D, _D), :], xv.at[slot], sems.at[slot])

    def _out_copy(o_hbm, ov, sems, t):
        return pltpu.make_async_copy(
            ov, o_hbm.at[pl.ds(t * _Q, _Q), :], sems.at[2])

    def body(x_hbm, o_hbm, xv, ov, sems):
        t = pl.program_id(0)
        slot = lax.rem(t, 2)

        @pl.when(t == 0)
        def _prologue():
            _in_copy(x_hbm, xv, sems, 0, 0).start()

        @pl.when(t < _NT - 1)
        def _prefetch():
            _in_copy(x_hbm, xv, sems, t + 1, 1 - slot).start()

        _in_copy(x_hbm, xv, sems, t, slot).wait()

        @pl.when(t >= 1)
        def _drain_prev():
            _out_copy(o_hbm, ov, sems, t - 1).wait()

        for j in range(4):
            xth = jnp.transpose(xv[slot, :, j * _Q:(j + 1) * _Q])
            ov[:, j * _D:(j + 1) * _D] = xth
        _out_copy(o_hbm, ov, sems, t).start()

        @pl.when(t >= _NT - 1)
        def _epilogue():
            _out_copy(o_hbm, ov, sems, t).wait()

    return pl.pallas_call(
        body,
        grid=(_NT,),
        in_specs=[pl.BlockSpec(memory_space=pl.ANY)],
        out_specs=pl.BlockSpec(memory_space=pl.ANY),
        out_shape=jax.ShapeDtypeStruct((_NT * _Q, 4 * _D), jnp.float32),
        scratch_shapes=[
            pltpu.VMEM((2, _D, _V), jnp.float32),
            pltpu.VMEM((_Q, 4 * _D), jnp.float32),
            pltpu.SemaphoreType.DMA((3,)),
        ],
        compiler_params=pltpu.CompilerParams(
            dimension_semantics=("arbitrary",),
            vmem_limit_bytes=60 * 1024 * 1024,
            fuse_transposed_lhs_in_matmul=True,
        ),
    )(embT2)


_BB = 1024


def _tc_body(numT, ly, bw0, bb0, bw1, bb1, bw2, bb2,
             tw0p, tb0, tw1, tb1, tw2, tb2, tw3, tb3, tw4, tb4, out):
    f32 = jnp.float32

    def mm(w, x):
        return jnp.dot(w, x, preferred_element_type=f32)

    x = numT[...]
    x = jnp.maximum(mm(bw0[...], x) + bb0[...], 0.0)
    x = jnp.maximum(mm(bw1[...], x) + bb1[...], 0.0)
    bot = jnp.maximum(mm(bw2[...], x) + bb2[...], 0.0)

    lyT = jnp.transpose(ly[...])
    ly3 = lyT.reshape(_NT, _D, _BB)

    pieces = [bot, jnp.sum(ly3 * bot[None, :, :], axis=1)]
    for o in range(1, _NT):
        kk = _NT - o
        v = ly3[o:, :, :] * ly3[:kk, :, :]
        pieces.append(jnp.sum(v, axis=1))
    t = jnp.concatenate(pieces, axis=0)

    t = jnp.maximum(mm(tw0p[...], t) + tb0[...], 0.0)
    t = jnp.maximum(mm(tw1[...], t) + tb1[...], 0.0)
    t = jnp.maximum(mm(tw2[...], t) + tb2[...], 0.0)
    t = jnp.maximum(mm(tw3[...], t) + tb3[...], 0.0)
    out[...] = jax.nn.sigmoid(mm(tw4[...], t) + tb4[...])


def _tc_forward(numT, ly2d, bw0, bb0, bw1, bb1, bw2, bb2,
                tw0p, tb0, tw1, tb1, tw2, tb2, tw3, tb3, tw4, tb4):
    nblk = _B // _BB

    def full(a):
        return pl.BlockSpec(a.shape, lambda i: (0,) * a.ndim)

    in_specs = [
        pl.BlockSpec((_ND, _BB), lambda i: (0, i)),
        pl.BlockSpec((_BB, _NT * _D), lambda i: (i, 0)),
    ] + [full(a) for a in (bw0, bb0, bw1, bb1, bw2, bb2,
                           tw0p, tb0, tw1, tb1, tw2, tb2, tw3, tb3, tw4, tb4)]

    return pl.pallas_call(
        _tc_body,
        grid=(nblk,),
        in_specs=in_specs,
        out_specs=pl.BlockSpec((1, _BB), lambda i: (0, i)),
        out_shape=jax.ShapeDtypeStruct((1, _B), jnp.float32),
        compiler_params=pltpu.CompilerParams(
            dimension_semantics=("arbitrary",),
        ),
    )(numT, ly2d, bw0, bb0, bw1, bb1, bw2, bb2,
      tw0p, tb0, tw1, tb1, tw2, tb2, tw3, tb3, tw4, tb4)


def _offset_perm():
    r_idx, c_idx = np.tril_indices(_N1, -1)
    pos = {(int(r), int(c)): j for j, (r, c) in enumerate(zip(r_idx, c_idx))}
    order = [pos[(o, 0)] for o in range(1, _N1)]
    order += [pos[(j + o + 1, j + 1)]
              for o in range(1, _NT) for j in range(_NT - o)]
    return np.array(order, dtype=np.int32)


def kernel(numerical_input, categorical_inputs, emb,
           bw0, bb0, bw1, bb1, bw2, bb2,
           tw0, tb0, tw1, tb1, tw2, tb2, tw3, tb3, tw4, tb4):
    embT2 = emb.transpose(0, 2, 1).reshape(_NT * _D, _V)
    cat = categorical_inputs
    shuf = 4 * (cat % _Q) + cat // _Q
    flat_idx = (shuf
                + (jnp.arange(_NT, dtype=jnp.int32) * _V)[None, :]).reshape(-1)
    numT = numerical_input.T
    perm = _offset_perm()
    tw0p = jnp.concatenate([tw0[:, :_D], tw0[:, _D + perm]], axis=1)

    def col(b):
        return b.reshape(-1, 1)

    table = _table_shuffle(embT2).reshape(_NT * _V, _D)

    ly = _sc_gather(table, flat_idx)
    ly2d = ly.reshape(_B, _NT * _D)

    outT = _tc_forward(numT, ly2d, bw0, col(bb0), bw1, col(bb1), bw2, col(bb2),
                       tw0p, col(tb0), tw1, col(tb1), tw2, col(tb2),
                       tw3, col(tb3), tw4, col(tb4))
    return outT.T
